# Initial kernel scaffold; baseline (speedup 1.0000x reference)
import jax
import jax.numpy as jnp
from jax.experimental import pallas as pl


def kernel(input, emb, Wih_f, Whh_f, bih_f, bhh_f, Wih_b, Whh_b, bih_b, bhh_b, fcW, fcb):
    raise NotImplementedError("write your pallas kernel here")


# speedup vs baseline: 1.0821x; 1.0821x over previous
import functools

import jax
import jax.numpy as jnp
from jax import lax
from jax.experimental import pallas as pl
from jax.experimental.pallas import tpu as pltpu
from jax.experimental.pallas import tpu_sc as plsc


_NC = 2
_NS = 16
_NW = _NC * _NS
_CHUNK = 80


def _sc_gather(emb, idx3d, n_idx, e_dim):
    rows_per_w = n_idx // _NW
    chunks_per_w = rows_per_w // _CHUNK
    mesh = plsc.VectorSubcoreMesh(core_axis_name="c", subcore_axis_name="s")

    @functools.partial(
        pl.kernel,
        mesh=mesh,
        out_type=jax.ShapeDtypeStruct((n_idx, e_dim), jnp.float32),
        scratch_types=[
            pltpu.VMEM((chunks_per_w, _CHUNK), jnp.int32),
            pltpu.VMEM((_CHUNK, e_dim), jnp.float32),
            pltpu.SemaphoreType.DMA,
        ],
    )
    def gather_kernel(emb_hbm, idx_hbm, out_hbm, idx_v, rows_v, sem):
        wid = lax.axis_index("s") * _NC + lax.axis_index("c")
        base = wid * rows_per_w
        pltpu.sync_copy(idx_hbm.at[wid], idx_v)
        for c in range(chunks_per_w):
            pltpu.async_copy(emb_hbm.at[idx_v.at[c]], rows_v, sem).wait()
            pltpu.sync_copy(rows_v, out_hbm.at[pl.ds(base + c * _CHUNK, _CHUNK)])

    return gather_kernel(emb, idx3d)


def _input_matmul_body(x_ref, w_ref, b_ref, o_ref):
    o_ref[...] = (
        jnp.dot(x_ref[...], w_ref[...], preferred_element_type=jnp.float32)
        + b_ref[...]
    )


def _input_matmul(x, wcat, bcat, bm=256):
    m, k = x.shape
    n = wcat.shape[1]
    return pl.pallas_call(
        _input_matmul_body,
        grid=(m // bm,),
        in_specs=[
            pl.BlockSpec((bm, k), lambda i: (i, 0)),
            pl.BlockSpec((k, n), lambda i: (0, 0)),
            pl.BlockSpec((1, n), lambda i: (0, 0)),
        ],
        out_specs=pl.BlockSpec((bm, n), lambda i: (i, 0)),
        out_shape=jax.ShapeDtypeStruct((m, n), jnp.float32),
        compiler_params=pltpu.CompilerParams(
            dimension_semantics=("arbitrary",),
        ),
    )(x, wcat, bcat)


def _gru_scan_body(gi_f_ref, gi_b_ref, whtf_ref, whtb_ref, bnf_ref, bnb_ref,
                   fcw_ref, fcb_ref, out_ref, hf_ref, hb_ref):
    t = pl.program_id(0)
    nsteps = pl.num_programs(0)

    @pl.when(t == 0)
    def _init():
        hf_ref[...] = jnp.zeros_like(hf_ref)
        hb_ref[...] = jnp.zeros_like(hb_ref)

    def step(gi_ref, wht_ref, bn_ref, h_ref):
        h = h_ref[...]
        hdim = h.shape[1]
        gi = gi_ref[0]
        gh = jnp.dot(h, wht_ref[...], preferred_element_type=jnp.float32)
        r = jax.nn.sigmoid(gi[:, :hdim] + gh[:, :hdim])
        z = jax.nn.sigmoid(gi[:, hdim:2 * hdim] + gh[:, hdim:2 * hdim])
        n = jnp.tanh(gi[:, 2 * hdim:] + r * (gh[:, 2 * hdim:] + bn_ref[...]))
        h_new = (1.0 - z) * n + z * h
        h_ref[...] = h_new
        return h_new

    hf = step(gi_f_ref, whtf_ref, bnf_ref, hf_ref)
    hb = step(gi_b_ref, whtb_ref, bnb_ref, hb_ref)

    @pl.when(t == nsteps - 1)
    def _head():
        hdim = hf.shape[1]
        wf = fcw_ref[0, :hdim][None, :]
        wb = fcw_ref[0, hdim:][None, :]
        logit = (jnp.sum(hf * wf, axis=1, keepdims=True)
                 + jnp.sum(hb * wb, axis=1, keepdims=True)
                 + fcb_ref[0, 0])
        out_ref[...] = jax.nn.sigmoid(logit)


def _gru_scan(gi, whtf, whtb, bnf, bnb, fcw, fcb, t_len, b_dim, h_dim):
    g3 = 3 * h_dim
    return pl.pallas_call(
        _gru_scan_body,
        grid=(t_len,),
        in_specs=[
            pl.BlockSpec((1, b_dim, g3), lambda t: (t, 0, 0)),
            pl.BlockSpec((1, b_dim, g3), lambda t: (t_len - 1 - t, 0, 1)),
            pl.BlockSpec((h_dim, g3), lambda t: (0, 0)),
            pl.BlockSpec((h_dim, g3), lambda t: (0, 0)),
            pl.BlockSpec((1, h_dim), lambda t: (0, 0)),
            pl.BlockSpec((1, h_dim), lambda t: (0, 0)),
            pl.BlockSpec((1, 2 * h_dim), lambda t: (0, 0)),
            pl.BlockSpec((1, 1), lambda t: (0, 0)),
        ],
        out_specs=pl.BlockSpec((b_dim, 1), lambda t: (0, 0)),
        out_shape=jax.ShapeDtypeStruct((b_dim, 1), jnp.float32),
        scratch_shapes=[
            pltpu.VMEM((b_dim, h_dim), jnp.float32),
            pltpu.VMEM((b_dim, h_dim), jnp.float32),
        ],
        compiler_params=pltpu.CompilerParams(
            dimension_semantics=("arbitrary",),
        ),
    )(gi, gi, whtf, whtb, bnf, bnb, fcw, fcb)


def kernel(input, emb, Wih_f, Whh_f, bih_f, bhh_f, Wih_b, Whh_b, bih_b, bhh_b,
           fcW, fcb):
    b_dim, t_len = input.shape
    v_dim, e_dim = emb.shape
    h_dim = Whh_f.shape[1]
    n_idx = b_dim * t_len

    e_pad = (e_dim + 127) // 128 * 128
    emb_p = jnp.pad(emb, ((0, 0), (0, e_pad - e_dim)))
    idx3d = input.T.reshape(_NW, n_idx // (_NW * _CHUNK), _CHUNK).astype(jnp.int32)
    x = _sc_gather(emb_p, idx3d, n_idx, e_pad)

    def fold_bias(bih, bhh):
        return jnp.concatenate(
            [bih[: 2 * h_dim] + bhh[: 2 * h_dim], bih[2 * h_dim:]])

    wcat = jnp.concatenate([Wih_f.T, Wih_b.T], axis=1)
    wcat = jnp.pad(wcat, ((0, e_pad - e_dim), (0, 0)))
    bcat = jnp.concatenate([fold_bias(bih_f, bhh_f),
                            fold_bias(bih_b, bhh_b)])[None, :]
    gi = _input_matmul(x, wcat, bcat)
    gi = gi.reshape(t_len, b_dim, 6 * h_dim)

    label = _gru_scan(
        gi,
        Whh_f.T, Whh_b.T,
        bhh_f[2 * h_dim:][None, :], bhh_b[2 * h_dim:][None, :],
        fcW, fcb.reshape(1, 1),
        t_len, b_dim, h_dim,
    )
    return jnp.squeeze(label, axis=1)
